# Initial kernel scaffold; baseline (speedup 1.0000x reference)
import jax
import jax.numpy as jnp
from jax.experimental import pallas as pl


def kernel(x, edge_index, batch, W, b):
    raise NotImplementedError("write your pallas kernel here")


# speedup vs baseline: 56.7820x; 56.7820x over previous
import functools

import jax
import jax.numpy as jnp
from jax import lax
from jax.experimental import pallas as pl
from jax.experimental.pallas import tpu as pltpu
from jax.experimental.pallas import tpu_sc as plsc

N_NODES = 100000
N_EDGES = 6400000
N_GRAPHS = 64

NC = 2
NS = 16
NW = NC * NS
L = 16

NP = 102400
PER_TILE_NP = NP // NS
PER_W_NP = NP // NW

K = 32
CHUNK = K * 128
F = 48
MAIN_E = NW * F * CHUNK
MAIN_ROWS_W = F * K
TAIL_E_W = (N_EDGES - MAIN_E) // NW
TAIL_CHUNKS = 2
TAIL_ROWS_W = TAIL_CHUNKS * K

_mesh = plsc.VectorSubcoreMesh(core_axis_name="c", subcore_axis_name="s")


@functools.partial(
    pl.kernel,
    out_type=jax.ShapeDtypeStruct((NC, NP), jnp.float32),
    mesh=_mesh,
    scratch_types=[
        pltpu.VMEM((CHUNK,), jnp.int32),
        pltpu.VMEM((CHUNK,), jnp.int32),
        pltpu.VMEM((CHUNK,), jnp.float32),
        pltpu.VMEM((PER_TILE_NP,), jnp.float32),
        pltpu.VMEM_SHARED((NP,), jnp.float32),
        pltpu.SemaphoreType.DMA,
        pltpu.SemaphoreType.DMA,
    ],
)
def _edge_kernel(edge_hbm, tsrc_hbm, tdst_hbm, x_hbm, aggp_hbm,
                 src_v, dst_v, vals_v, stage_v, agg_sh, gsem, ssem):
    cid = lax.axis_index("c")
    sid = lax.axis_index("s")
    wid = cid * NS + sid

    def _z(i, _):
        stage_v[pl.ds(i * L, L)] = jnp.zeros((L,), jnp.float32)
        return 0
    lax.fori_loop(0, PER_TILE_NP // L, _z, 0)
    pltpu.sync_copy(stage_v, agg_sh.at[pl.ds(sid * PER_TILE_NP, PER_TILE_NP)])
    plsc.subcore_barrier()

    def do_chunk(src_slice, dst_slice):
        pltpu.sync_copy(src_slice, src_v)
        pltpu.sync_copy(dst_slice, dst_v)
        pltpu.async_copy(x_hbm.at[src_v], vals_v, gsem).wait()
        pltpu.async_copy(vals_v, agg_sh.at[dst_v], ssem, add=True).wait()

    def _main(t, _):
        e0 = (wid * F + t) * CHUNK
        do_chunk(edge_hbm.at[0, pl.ds(e0, CHUNK)], edge_hbm.at[1, pl.ds(e0, CHUNK)])
        return 0
    lax.fori_loop(0, F, _main, 0)

    for t in range(TAIL_CHUNKS):
        e0 = (wid * TAIL_CHUNKS + t) * CHUNK
        do_chunk(tsrc_hbm.at[pl.ds(e0, CHUNK)], tdst_hbm.at[pl.ds(e0, CHUNK)])

    plsc.subcore_barrier()
    pltpu.sync_copy(agg_sh.at[pl.ds(sid * PER_TILE_NP, PER_TILE_NP)], stage_v)
    pltpu.sync_copy(stage_v, aggp_hbm.at[cid, pl.ds(sid * PER_TILE_NP, PER_TILE_NP)])


R2 = PER_W_NP // 128


@functools.partial(
    pl.kernel,
    out_type=(jax.ShapeDtypeStruct((NC, 128), jnp.float32),
              jax.ShapeDtypeStruct((NC, 128), jnp.float32)),
    mesh=_mesh,
    scratch_types=[
        pltpu.VMEM((PER_W_NP,), jnp.float32),
        pltpu.VMEM((PER_W_NP,), jnp.float32),
        pltpu.VMEM((PER_W_NP,), jnp.float32),
        pltpu.VMEM((PER_W_NP,), jnp.float32),
        pltpu.VMEM((PER_W_NP,), jnp.int32),
        pltpu.VMEM((128,), jnp.float32),
        pltpu.VMEM_SHARED((128,), jnp.float32),
        pltpu.VMEM_SHARED((128,), jnp.float32),
        pltpu.SemaphoreType.DMA,
        pltpu.SemaphoreType.DMA,
    ],
)
def _pool_kernel(aggp_hbm, batch_hbm, sums_hbm, cnts_hbm,
                 a0_v, a1_v, h_v, ones_v, bidx_v, stage_v,
                 sums_sh, cnts_sh, hsem, csem):
    cid = lax.axis_index("c")
    sid = lax.axis_index("s")
    wid = cid * NS + sid

    for i in range(128 // L):
        stage_v[pl.ds(i * L, L)] = jnp.zeros((L,), jnp.float32)

    @pl.when(sid == 0)
    def _():
        pltpu.sync_copy(stage_v, sums_sh)
        pltpu.sync_copy(stage_v, cnts_sh)
    plsc.subcore_barrier()

    base = wid * PER_W_NP
    pltpu.sync_copy(aggp_hbm.at[0, pl.ds(base, PER_W_NP)], a0_v)
    pltpu.sync_copy(aggp_hbm.at[1, pl.ds(base, PER_W_NP)], a1_v)
    pltpu.sync_copy(batch_hbm.at[pl.ds(base, PER_W_NP)], bidx_v)

    def _relu(i, _):
        off = i * L
        a = a0_v[pl.ds(off, L)] + a1_v[pl.ds(off, L)]
        h_v[pl.ds(off, L)] = jnp.maximum(a, 0.0)
        ones_v[pl.ds(off, L)] = jnp.ones((L,), jnp.float32)
        return 0
    lax.fori_loop(0, PER_W_NP // L, _relu, 0)

    pltpu.async_copy(h_v, sums_sh.at[bidx_v], hsem, add=True).wait()
    pltpu.async_copy(ones_v, cnts_sh.at[bidx_v], csem, add=True).wait()

    plsc.subcore_barrier()

    @pl.when(sid == 0)
    def _():
        pltpu.sync_copy(sums_sh, stage_v)
        pltpu.sync_copy(stage_v, sums_hbm.at[cid])
        pltpu.sync_copy(cnts_sh, stage_v)
        pltpu.sync_copy(stage_v, cnts_hbm.at[cid])


def kernel(x, edge_index, batch, W, b):
    ei = edge_index.astype(jnp.int32)
    edge3 = ei[:, :MAIN_E]
    tail = ei[:, MAIN_E:].reshape(2, NW, TAIL_E_W)
    pad = jnp.full((2, NW, TAIL_CHUNKS * CHUNK - TAIL_E_W), N_NODES, jnp.int32)
    tailp = jnp.concatenate([tail, pad], axis=2)
    tsrc = tailp[0].reshape(NW * TAIL_CHUNKS * CHUNK)
    tdst = tailp[1].reshape(NW * TAIL_CHUNKS * CHUNK)

    x_p = jnp.concatenate([x.reshape(-1), jnp.zeros((NP - N_NODES,), jnp.float32)])
    batch_p = jnp.concatenate(
        [batch.astype(jnp.int32),
         jnp.full((NP - N_NODES,), N_GRAPHS, jnp.int32)])

    aggp = _edge_kernel(edge3, tsrc, tdst, x_p)
    sums, cnts = _pool_kernel(aggp, batch_p)

    tot_s = sums[0] + sums[1]
    tot_c = cnts[0] + cnts[1]
    pooled = tot_s[:N_GRAPHS] / jnp.maximum(tot_c[:N_GRAPHS], 1.0)
    return pooled[:, None] * W[0, 0] + b


# speedup vs baseline: 170.7169x; 3.0065x over previous
import functools

import jax
import jax.numpy as jnp
from jax import lax
from jax.experimental import pallas as pl
from jax.experimental.pallas import tpu as pltpu
from jax.experimental.pallas import tpu_sc as plsc

N_NODES = 100000
N_EDGES = 6400000
N_GRAPHS = 64

NC = 2
NS = 16
NW = NC * NS
L = 16

NP = 102400
PER_TILE_NP = NP // NS
PER_W_NP = NP // NW

CHUNK = 8192
NCH = N_EDGES // CHUNK
TAIL = N_EDGES - NCH * CHUNK
PAIRS = 12
EXTRA = NCH - NW * 2 * PAIRS

_mesh = plsc.VectorSubcoreMesh(core_axis_name="c", subcore_axis_name="s")


@functools.partial(
    pl.kernel,
    out_type=jax.ShapeDtypeStruct((NC, NP), jnp.float32),
    mesh=_mesh,
    scratch_types=[
        pltpu.VMEM((CHUNK,), jnp.int32),
        pltpu.VMEM((CHUNK,), jnp.int32),
        pltpu.VMEM((CHUNK,), jnp.float32),
        pltpu.VMEM((CHUNK,), jnp.int32),
        pltpu.VMEM((CHUNK,), jnp.int32),
        pltpu.VMEM((CHUNK,), jnp.float32),
        pltpu.VMEM((TAIL,), jnp.int32),
        pltpu.VMEM((TAIL,), jnp.int32),
        pltpu.VMEM((TAIL,), jnp.float32),
        pltpu.VMEM((PER_TILE_NP,), jnp.float32),
        pltpu.VMEM_SHARED((NP,), jnp.float32),
        pltpu.SemaphoreType.DMA,
        pltpu.SemaphoreType.DMA,
        pltpu.SemaphoreType.DMA,
        pltpu.SemaphoreType.DMA,
    ],
)
def _edge_kernel(edge_hbm, x_hbm, aggp_hbm,
                 srcA, dstA, valsA, srcB, dstB, valsB, srcT, dstT, valsT,
                 stage_v, agg_sh, gsemA, ssemA, gsemB, ssemB):
    cid = lax.axis_index("c")
    sid = lax.axis_index("s")
    wid = sid * NC + cid

    def _z(i, _):
        stage_v[pl.ds(i * L, L)] = jnp.zeros((L,), jnp.float32)
        return 0
    lax.fori_loop(0, PER_TILE_NP // L, _z, 0)
    pltpu.sync_copy(stage_v, agg_sh.at[pl.ds(sid * PER_TILE_NP, PER_TILE_NP)])
    plsc.subcore_barrier()

    start = wid * 2 * PAIRS + jnp.minimum(wid, EXTRA)

    def fetch(c, src_v, dst_v, vals_v, gsem):
        e0 = c * CHUNK
        pltpu.sync_copy(edge_hbm.at[0, pl.ds(e0, CHUNK)], src_v)
        pltpu.sync_copy(edge_hbm.at[1, pl.ds(e0, CHUNK)], dst_v)
        return pltpu.async_copy(x_hbm.at[src_v], vals_v, gsem)

    def _pair(t, _):
        a = start + 2 * t
        ga = fetch(a, srcA, dstA, valsA, gsemA)
        gb = fetch(a + 1, srcB, dstB, valsB, gsemB)
        ga.wait()
        sa = pltpu.async_copy(valsA, agg_sh.at[dstA], ssemA, add=True)
        gb.wait()
        sb = pltpu.async_copy(valsB, agg_sh.at[dstB], ssemB, add=True)
        sa.wait()
        sb.wait()
        return 0
    lax.fori_loop(0, PAIRS, _pair, 0)

    @pl.when(wid < EXTRA)
    def _():
        ga = fetch(start + 2 * PAIRS, srcA, dstA, valsA, gsemA)
        ga.wait()
        pltpu.async_copy(valsA, agg_sh.at[dstA], ssemA, add=True).wait()

    @pl.when(wid == NW - 1)
    def _():
        e0 = NCH * CHUNK
        pltpu.sync_copy(edge_hbm.at[0, pl.ds(e0, TAIL)], srcT)
        pltpu.sync_copy(edge_hbm.at[1, pl.ds(e0, TAIL)], dstT)
        pltpu.async_copy(x_hbm.at[srcT], valsT, gsemB).wait()
        pltpu.async_copy(valsT, agg_sh.at[dstT], ssemB, add=True).wait()

    plsc.subcore_barrier()
    pltpu.sync_copy(agg_sh.at[pl.ds(sid * PER_TILE_NP, PER_TILE_NP)], stage_v)
    pltpu.sync_copy(stage_v, aggp_hbm.at[cid, pl.ds(sid * PER_TILE_NP, PER_TILE_NP)])


@functools.partial(
    pl.kernel,
    out_type=(jax.ShapeDtypeStruct((NC, 128), jnp.float32),
              jax.ShapeDtypeStruct((NC, 128), jnp.float32)),
    mesh=_mesh,
    scratch_types=[
        pltpu.VMEM((PER_W_NP,), jnp.float32),
        pltpu.VMEM((PER_W_NP,), jnp.float32),
        pltpu.VMEM((PER_W_NP,), jnp.float32),
        pltpu.VMEM((PER_W_NP,), jnp.float32),
        pltpu.VMEM((PER_W_NP,), jnp.int32),
        pltpu.VMEM((128,), jnp.float32),
        pltpu.VMEM_SHARED((128,), jnp.float32),
        pltpu.VMEM_SHARED((128,), jnp.float32),
        pltpu.SemaphoreType.DMA,
        pltpu.SemaphoreType.DMA,
    ],
)
def _pool_kernel(aggp_hbm, batch_hbm, sums_hbm, cnts_hbm,
                 a0_v, a1_v, h_v, ones_v, bidx_v, stage_v,
                 sums_sh, cnts_sh, hsem, csem):
    cid = lax.axis_index("c")
    sid = lax.axis_index("s")
    wid = cid * NS + sid

    for i in range(128 // L):
        stage_v[pl.ds(i * L, L)] = jnp.zeros((L,), jnp.float32)

    @pl.when(sid == 0)
    def _():
        pltpu.sync_copy(stage_v, sums_sh)
        pltpu.sync_copy(stage_v, cnts_sh)
    plsc.subcore_barrier()

    base = wid * PER_W_NP
    pltpu.sync_copy(aggp_hbm.at[0, pl.ds(base, PER_W_NP)], a0_v)
    pltpu.sync_copy(aggp_hbm.at[1, pl.ds(base, PER_W_NP)], a1_v)
    pltpu.sync_copy(batch_hbm.at[pl.ds(base, PER_W_NP)], bidx_v)

    def _relu(i, _):
        off = i * L
        a = a0_v[pl.ds(off, L)] + a1_v[pl.ds(off, L)]
        h_v[pl.ds(off, L)] = jnp.maximum(a, 0.0)
        ones_v[pl.ds(off, L)] = jnp.ones((L,), jnp.float32)
        return 0
    lax.fori_loop(0, PER_W_NP // L, _relu, 0)

    pltpu.async_copy(h_v, sums_sh.at[bidx_v], hsem, add=True).wait()
    pltpu.async_copy(ones_v, cnts_sh.at[bidx_v], csem, add=True).wait()

    plsc.subcore_barrier()

    @pl.when(sid == 0)
    def _():
        pltpu.sync_copy(sums_sh, stage_v)
        pltpu.sync_copy(stage_v, sums_hbm.at[cid])
        pltpu.sync_copy(cnts_sh, stage_v)
        pltpu.sync_copy(stage_v, cnts_hbm.at[cid])


def kernel(x, edge_index, batch, W, b):
    ei = edge_index.astype(jnp.int32)
    batch_p = jnp.concatenate(
        [batch.astype(jnp.int32),
         jnp.full((NP - N_NODES,), N_GRAPHS, jnp.int32)])

    aggp = _edge_kernel(ei, x.reshape(-1))
    sums, cnts = _pool_kernel(aggp, batch_p)

    tot_s = sums[0] + sums[1]
    tot_c = cnts[0] + cnts[1]
    pooled = tot_s[:N_GRAPHS] / jnp.maximum(tot_c[:N_GRAPHS], 1.0)
    return pooled[:, None] * W[0, 0] + b


# speedup vs baseline: 475.7932x; 2.7870x over previous
import functools

import jax
import jax.numpy as jnp
from jax import lax
from jax.experimental import pallas as pl
from jax.experimental.pallas import tpu as pltpu
from jax.experimental.pallas import tpu_sc as plsc

N_NODES = 100000
N_EDGES = 6400000
N_GRAPHS = 64

NC = 2
NS = 16
NW = NC * NS
L = 16

NP = 102400
PER_TILE_NP = NP // NS
PER_W_NP = NP // NW

CHUNK = 2048
NCH = N_EDGES // CHUNK
PAIRS = 48
CNT = 97
EXTRA2 = NCH - NW * CNT
U = 8

_mesh = plsc.VectorSubcoreMesh(core_axis_name="c", subcore_axis_name="s")


@functools.partial(
    pl.kernel,
    out_type=jax.ShapeDtypeStruct((NC, NP), jnp.float32),
    mesh=_mesh,
    scratch_types=[
        pltpu.VMEM((N_NODES,), jnp.float32),
        pltpu.VMEM((CHUNK,), jnp.int32),
        pltpu.VMEM((CHUNK,), jnp.int32),
        pltpu.VMEM((CHUNK,), jnp.float32),
        pltpu.VMEM((CHUNK,), jnp.int32),
        pltpu.VMEM((CHUNK,), jnp.int32),
        pltpu.VMEM((CHUNK,), jnp.float32),
        pltpu.VMEM_SHARED((NP,), jnp.float32),
        pltpu.SemaphoreType.DMA,
        pltpu.SemaphoreType.DMA,
        pltpu.SemaphoreType.DMA,
        pltpu.SemaphoreType.DMA,
        pltpu.SemaphoreType.DMA,
    ],
    compiler_params=pltpu.CompilerParams(needs_layout_passes=False),
)
def _edge_kernel(edge_hbm, x_hbm, aggp_hbm,
                 x_v, srcA, dstA, valsA, srcB, dstB, valsB,
                 agg_sh, xsem, isemA, ssemA, isemB, ssemB):
    cid = lax.axis_index("c")
    sid = lax.axis_index("s")
    wid = sid * NC + cid

    xcopy = pltpu.async_copy(x_hbm, x_v, xsem)

    def _z(i, _):
        valsA[pl.ds(i * L, L)] = jnp.zeros((L,), jnp.float32)
        return 0
    lax.fori_loop(0, CHUNK // L, _z, 0)
    z0 = sid * PER_TILE_NP
    for p in range(PER_TILE_NP // CHUNK):
        pltpu.sync_copy(valsA, agg_sh.at[pl.ds(z0 + p * CHUNK, CHUNK)])
    _rem = PER_TILE_NP - (PER_TILE_NP // CHUNK) * CHUNK
    pltpu.sync_copy(valsA.at[pl.ds(0, _rem)],
                    agg_sh.at[pl.ds(z0 + PER_TILE_NP - _rem, _rem)])

    start = wid * CNT + jnp.minimum(wid, EXTRA2)

    def issue_idx(c, src_v, dst_v, isem, n=CHUNK):
        e0 = c * CHUNK
        pltpu.async_copy(edge_hbm.at[0, pl.ds(e0, n)], src_v, isem)
        pltpu.async_copy(edge_hbm.at[1, pl.ds(e0, n)], dst_v, isem)

    def wait_idx(c, src_v, dst_v, isem, n=CHUNK):
        e0 = c * CHUNK
        pltpu.make_async_copy(edge_hbm.at[0, pl.ds(e0, n)], src_v, isem).wait()
        pltpu.make_async_copy(edge_hbm.at[1, pl.ds(e0, n)], dst_v, isem).wait()

    def gather_loop(src_v, vals_v, n=CHUNK):
        def _g(i, _):
            for u in range(U):
                off = i * (L * U) + u * L
                idx = src_v[pl.ds(off, L)]
                vals_v[pl.ds(off, L)] = plsc.load_gather(x_v, [idx])
            return 0
        lax.fori_loop(0, n // (L * U), _g, 0)

    issue_idx(start, srcA, dstA, isemA)
    issue_idx(start + 1, srcB, dstB, isemB)
    xcopy.wait()
    plsc.subcore_barrier()

    def _pair(t, _):
        a = start + 2 * t
        wait_idx(a, srcA, dstA, isemA)
        gather_loop(srcA, valsA)
        sa = pltpu.async_copy(valsA, agg_sh.at[dstA], ssemA, add=True)
        wait_idx(a + 1, srcB, dstB, isemB)
        gather_loop(srcB, valsB)
        sb = pltpu.async_copy(valsB, agg_sh.at[dstB], ssemB, add=True)
        sa.wait()

        @pl.when(t < PAIRS - 1)
        def _():
            issue_idx(a + 2, srcA, dstA, isemA)

        @pl.when(t == PAIRS - 1)
        def _():
            issue_idx(start + 2 * PAIRS, srcA, dstA, isemA)
        sb.wait()

        @pl.when(t < PAIRS - 1)
        def _():
            issue_idx(a + 3, srcB, dstB, isemB)

        @pl.when(jnp.logical_and(t == PAIRS - 1, wid < EXTRA2))
        def _():
            issue_idx(start + 2 * PAIRS + 1, srcB, dstB, isemB)
        return 0
    lax.fori_loop(0, PAIRS, _pair, 0)

    c1 = start + 2 * PAIRS
    wait_idx(c1, srcA, dstA, isemA)
    gather_loop(srcA, valsA)
    sc1 = pltpu.async_copy(valsA, agg_sh.at[dstA], ssemA, add=True)

    @pl.when(wid < EXTRA2)
    def _():
        wait_idx(c1 + 1, srcB, dstB, isemB)
        gather_loop(srcB, valsB)
        pltpu.async_copy(valsB, agg_sh.at[dstB], ssemB, add=True).wait()
    sc1.wait()

    plsc.subcore_barrier()
    for p in range(PER_TILE_NP // CHUNK):
        pltpu.sync_copy(agg_sh.at[pl.ds(z0 + p * CHUNK, CHUNK)], valsA)
        pltpu.sync_copy(valsA, aggp_hbm.at[cid, pl.ds(z0 + p * CHUNK, CHUNK)])
    pltpu.sync_copy(agg_sh.at[pl.ds(z0 + PER_TILE_NP - _rem, _rem)],
                    valsA.at[pl.ds(0, _rem)])
    pltpu.sync_copy(valsA.at[pl.ds(0, _rem)],
                    aggp_hbm.at[cid, pl.ds(z0 + PER_TILE_NP - _rem, _rem)])


@functools.partial(
    pl.kernel,
    out_type=(jax.ShapeDtypeStruct((NC, 128), jnp.float32),
              jax.ShapeDtypeStruct((NC, 128), jnp.float32)),
    mesh=_mesh,
    scratch_types=[
        pltpu.VMEM((PER_W_NP,), jnp.float32),
        pltpu.VMEM((PER_W_NP,), jnp.float32),
        pltpu.VMEM((PER_W_NP,), jnp.float32),
        pltpu.VMEM((PER_W_NP,), jnp.float32),
        pltpu.VMEM((PER_W_NP,), jnp.int32),
        pltpu.VMEM((128,), jnp.float32),
        pltpu.VMEM_SHARED((128,), jnp.float32),
        pltpu.VMEM_SHARED((128,), jnp.float32),
        pltpu.SemaphoreType.DMA,
        pltpu.SemaphoreType.DMA,
    ],
)
def _pool_kernel(aggp_hbm, batch_hbm, sums_hbm, cnts_hbm,
                 a0_v, a1_v, h_v, ones_v, bidx_v, stage_v,
                 sums_sh, cnts_sh, hsem, csem):
    cid = lax.axis_index("c")
    sid = lax.axis_index("s")
    wid = cid * NS + sid

    for i in range(128 // L):
        stage_v[pl.ds(i * L, L)] = jnp.zeros((L,), jnp.float32)

    @pl.when(sid == 0)
    def _():
        pltpu.sync_copy(stage_v, sums_sh)
        pltpu.sync_copy(stage_v, cnts_sh)
    plsc.subcore_barrier()

    base = wid * PER_W_NP
    pltpu.sync_copy(aggp_hbm.at[0, pl.ds(base, PER_W_NP)], a0_v)
    pltpu.sync_copy(aggp_hbm.at[1, pl.ds(base, PER_W_NP)], a1_v)
    pltpu.sync_copy(batch_hbm.at[pl.ds(base, PER_W_NP)], bidx_v)

    def _relu(i, _):
        off = i * L
        a = a0_v[pl.ds(off, L)] + a1_v[pl.ds(off, L)]
        h_v[pl.ds(off, L)] = jnp.maximum(a, 0.0)
        ones_v[pl.ds(off, L)] = jnp.ones((L,), jnp.float32)
        return 0
    lax.fori_loop(0, PER_W_NP // L, _relu, 0)

    pltpu.async_copy(h_v, sums_sh.at[bidx_v], hsem, add=True).wait()
    pltpu.async_copy(ones_v, cnts_sh.at[bidx_v], csem, add=True).wait()

    plsc.subcore_barrier()

    @pl.when(sid == 0)
    def _():
        pltpu.sync_copy(sums_sh, stage_v)
        pltpu.sync_copy(stage_v, sums_hbm.at[cid])
        pltpu.sync_copy(cnts_sh, stage_v)
        pltpu.sync_copy(stage_v, cnts_hbm.at[cid])


def kernel(x, edge_index, batch, W, b):
    ei = edge_index.astype(jnp.int32)
    batch_p = jnp.concatenate(
        [batch.astype(jnp.int32),
         jnp.full((NP - N_NODES,), N_GRAPHS, jnp.int32)])

    aggp = _edge_kernel(ei, x.reshape(-1))
    sums, cnts = _pool_kernel(aggp, batch_p)

    tot_s = sums[0] + sums[1]
    tot_c = cnts[0] + cnts[1]
    pooled = tot_s[:N_GRAPHS] / jnp.maximum(tot_c[:N_GRAPHS], 1.0)
    return pooled[:, None] * W[0, 0] + b


# speedup vs baseline: 641.1738x; 1.3476x over previous
import functools

import jax
import jax.numpy as jnp
from jax import lax
from jax.experimental import pallas as pl
from jax.experimental.pallas import tpu as pltpu
from jax.experimental.pallas import tpu_sc as plsc

N_NODES = 100000
N_EDGES = 6400000
N_GRAPHS = 64

NC = 2
NS = 16
NW = NC * NS
L = 16

NP = 102400
PER_TILE_NP = NP // NS
PER_W_NP = NP // NW

CHUNK = 2048
NCH = N_EDGES // CHUNK
CNT = 97
EXTRA2 = NCH - NW * CNT
NBUF = 4
LA = 3
GROUPS = 24
U = 8

_mesh = plsc.VectorSubcoreMesh(core_axis_name="c", subcore_axis_name="s")


@functools.partial(
    pl.kernel,
    out_type=jax.ShapeDtypeStruct((NC, NP), jnp.float32),
    mesh=_mesh,
    scratch_types=[
        pltpu.VMEM((N_NODES,), jnp.float32),
    ] + [pltpu.VMEM((CHUNK,), jnp.int32) for _ in range(NBUF)]
      + [pltpu.VMEM((CHUNK,), jnp.int32) for _ in range(NBUF)]
      + [pltpu.VMEM((CHUNK,), jnp.float32) for _ in range(NBUF)]
      + [pltpu.VMEM_SHARED((NP,), jnp.float32)]
      + [pltpu.SemaphoreType.DMA]
      + [pltpu.SemaphoreType.DMA for _ in range(NBUF)]
      + [pltpu.SemaphoreType.DMA for _ in range(NBUF)],
    compiler_params=pltpu.CompilerParams(needs_layout_passes=False),
)
def _edge_kernel(edge_hbm, x_hbm, aggp_hbm, x_v, *rest):
    src = rest[0:NBUF]
    dst = rest[NBUF:2 * NBUF]
    vals = rest[2 * NBUF:3 * NBUF]
    agg_sh = rest[3 * NBUF]
    xsem = rest[3 * NBUF + 1]
    isem = rest[3 * NBUF + 2:3 * NBUF + 2 + NBUF]
    ssem = rest[3 * NBUF + 2 + NBUF:3 * NBUF + 2 + 2 * NBUF]
    cid = lax.axis_index("c")
    sid = lax.axis_index("s")
    wid = sid * NC + cid

    xcopy = pltpu.async_copy(x_hbm, x_v, xsem)

    valsA = vals[0]

    def _z(i, _):
        valsA[pl.ds(i * L, L)] = jnp.zeros((L,), jnp.float32)
        return 0
    lax.fori_loop(0, CHUNK // L, _z, 0)
    z0 = sid * PER_TILE_NP
    for p in range(PER_TILE_NP // CHUNK):
        pltpu.sync_copy(valsA, agg_sh.at[pl.ds(z0 + p * CHUNK, CHUNK)])
    _rem = PER_TILE_NP - (PER_TILE_NP // CHUNK) * CHUNK
    pltpu.sync_copy(valsA.at[pl.ds(0, _rem)],
                    agg_sh.at[pl.ds(z0 + PER_TILE_NP - _rem, _rem)])

    start = wid * CNT + jnp.minimum(wid, EXTRA2)

    cnt_w = jnp.where(wid < EXTRA2, CNT + 1, CNT)

    def issue_idx(c, b):
        e0 = c * CHUNK
        pltpu.async_copy(edge_hbm.at[0, pl.ds(e0, CHUNK)], src[b], isem[b])
        pltpu.async_copy(edge_hbm.at[1, pl.ds(e0, CHUNK)], dst[b], isem[b])

    def wait_idx(c, b):
        e0 = c * CHUNK
        pltpu.make_async_copy(edge_hbm.at[0, pl.ds(e0, CHUNK)], src[b], isem[b]).wait()
        pltpu.make_async_copy(edge_hbm.at[1, pl.ds(e0, CHUNK)], dst[b], isem[b]).wait()

    def start_scatter(b):
        return pltpu.async_copy(vals[b], agg_sh.at[dst[b]], ssem[b], add=True)

    def drain_scatter(b):
        pltpu.make_async_copy(vals[b], agg_sh.at[dst[b]], ssem[b]).wait()

    def gather_loop(b):
        src_v, vals_v = src[b], vals[b]

        def _g(i, _):
            for u in range(U):
                off = i * (L * U) + u * L
                idx = src_v[pl.ds(off, L)]
                vals_v[pl.ds(off, L)] = plsc.load_gather(x_v, [idx])
            return 0
        lax.fori_loop(0, CHUNK // (L * U), _g, 0)

    for b in range(LA):
        issue_idx(start + b, b)
    xcopy.wait()
    plsc.subcore_barrier()

    def _group(g, _):
        j0 = NBUF * g
        for b in range(NBUF):
            j = j0 + b
            c = start + j
            wait_idx(c, b)
            gather_loop(b)
            start_scatter(b)
            pb = (b + LA) % NBUF
            if b == 0:
                @pl.when(g > 0)
                def _():
                    drain_scatter(pb)
            else:
                drain_scatter(pb)

            @pl.when(j + LA < cnt_w)
            def _():
                issue_idx(c + LA, pb)
        return 0
    lax.fori_loop(0, GROUPS, _group, 0)

    j96 = NBUF * GROUPS
    wait_idx(start + j96, 0)
    gather_loop(0)
    start_scatter(0)

    @pl.when(wid < EXTRA2)
    def _():
        wait_idx(start + j96 + 1, 1)
        gather_loop(1)
        start_scatter(1).wait()
    drain_scatter(NBUF - 1)
    drain_scatter(0)

    plsc.subcore_barrier()
    for p in range(PER_TILE_NP // CHUNK):
        pltpu.sync_copy(agg_sh.at[pl.ds(z0 + p * CHUNK, CHUNK)], valsA)
        pltpu.sync_copy(valsA, aggp_hbm.at[cid, pl.ds(z0 + p * CHUNK, CHUNK)])
    pltpu.sync_copy(agg_sh.at[pl.ds(z0 + PER_TILE_NP - _rem, _rem)],
                    valsA.at[pl.ds(0, _rem)])
    pltpu.sync_copy(valsA.at[pl.ds(0, _rem)],
                    aggp_hbm.at[cid, pl.ds(z0 + PER_TILE_NP - _rem, _rem)])


@functools.partial(
    pl.kernel,
    out_type=(jax.ShapeDtypeStruct((NC, 128), jnp.float32),
              jax.ShapeDtypeStruct((NC, 128), jnp.float32)),
    mesh=_mesh,
    scratch_types=[
        pltpu.VMEM((PER_W_NP,), jnp.float32),
        pltpu.VMEM((PER_W_NP,), jnp.float32),
        pltpu.VMEM((PER_W_NP,), jnp.float32),
        pltpu.VMEM((PER_W_NP,), jnp.float32),
        pltpu.VMEM((PER_W_NP,), jnp.int32),
        pltpu.VMEM((128,), jnp.float32),
        pltpu.VMEM_SHARED((128,), jnp.float32),
        pltpu.VMEM_SHARED((128,), jnp.float32),
        pltpu.SemaphoreType.DMA,
        pltpu.SemaphoreType.DMA,
    ],
)
def _pool_kernel(aggp_hbm, batch_hbm, sums_hbm, cnts_hbm,
                 a0_v, a1_v, h_v, ones_v, bidx_v, stage_v,
                 sums_sh, cnts_sh, hsem, csem):
    cid = lax.axis_index("c")
    sid = lax.axis_index("s")
    wid = cid * NS + sid

    for i in range(128 // L):
        stage_v[pl.ds(i * L, L)] = jnp.zeros((L,), jnp.float32)

    @pl.when(sid == 0)
    def _():
        pltpu.sync_copy(stage_v, sums_sh)
        pltpu.sync_copy(stage_v, cnts_sh)
    plsc.subcore_barrier()

    base = wid * PER_W_NP
    pltpu.sync_copy(aggp_hbm.at[0, pl.ds(base, PER_W_NP)], a0_v)
    pltpu.sync_copy(aggp_hbm.at[1, pl.ds(base, PER_W_NP)], a1_v)
    pltpu.sync_copy(batch_hbm.at[pl.ds(base, PER_W_NP)], bidx_v)

    def _relu(i, _):
        off = i * L
        a = a0_v[pl.ds(off, L)] + a1_v[pl.ds(off, L)]
        h_v[pl.ds(off, L)] = jnp.maximum(a, 0.0)
        ones_v[pl.ds(off, L)] = jnp.ones((L,), jnp.float32)
        return 0
    lax.fori_loop(0, PER_W_NP // L, _relu, 0)

    pltpu.async_copy(h_v, sums_sh.at[bidx_v], hsem, add=True).wait()
    pltpu.async_copy(ones_v, cnts_sh.at[bidx_v], csem, add=True).wait()

    plsc.subcore_barrier()

    @pl.when(sid == 0)
    def _():
        pltpu.sync_copy(sums_sh, stage_v)
        pltpu.sync_copy(stage_v, sums_hbm.at[cid])
        pltpu.sync_copy(cnts_sh, stage_v)
        pltpu.sync_copy(stage_v, cnts_hbm.at[cid])


def kernel(x, edge_index, batch, W, b):
    ei = edge_index.astype(jnp.int32)
    batch_p = jnp.concatenate(
        [batch.astype(jnp.int32),
         jnp.full((NP - N_NODES,), N_GRAPHS, jnp.int32)])

    aggp = _edge_kernel(ei, x.reshape(-1))
    sums, cnts = _pool_kernel(aggp, batch_p)

    tot_s = sums[0] + sums[1]
    tot_c = cnts[0] + cnts[1]
    pooled = tot_s[:N_GRAPHS] / jnp.maximum(tot_c[:N_GRAPHS], 1.0)
    return pooled[:, None] * W[0, 0] + b


# speedup vs baseline: 642.6949x; 1.0024x over previous
import functools

import jax
import jax.numpy as jnp
from jax import lax
from jax.experimental import pallas as pl
from jax.experimental.pallas import tpu as pltpu
from jax.experimental.pallas import tpu_sc as plsc

N_NODES = 100000
N_EDGES = 6400000
N_GRAPHS = 64

NC = 2
NS = 16
NW = NC * NS
L = 16

NP = 102400
PER_TILE_NP = NP // NS
PER_W_NP = NP // NW

CHUNK = 2048
NCH = N_EDGES // CHUNK
CNT = 97
EXTRA2 = NCH - NW * CNT
NBUF = 4
LA = 3
GROUPS = 24
U = 8

_mesh = plsc.VectorSubcoreMesh(core_axis_name="c", subcore_axis_name="s")


@functools.partial(
    pl.kernel,
    out_type=jax.ShapeDtypeStruct((NC, NP), jnp.float32),
    mesh=_mesh,
    scratch_types=[
        pltpu.VMEM((N_NODES,), jnp.float32),
    ] + [pltpu.VMEM((CHUNK,), jnp.int32) for _ in range(NBUF)]
      + [pltpu.VMEM((CHUNK,), jnp.int32) for _ in range(NBUF)]
      + [pltpu.VMEM((CHUNK,), jnp.float32) for _ in range(NBUF)]
      + [pltpu.VMEM_SHARED((NP,), jnp.float32)]
      + [pltpu.SemaphoreType.DMA]
      + [pltpu.SemaphoreType.DMA for _ in range(NBUF)]
      + [pltpu.SemaphoreType.DMA for _ in range(NBUF)],
    compiler_params=pltpu.CompilerParams(needs_layout_passes=False),
)
def _edge_kernel(edge_hbm, x_hbm, aggp_hbm, x_v, *rest):
    src = rest[0:NBUF]
    dst = rest[NBUF:2 * NBUF]
    vals = rest[2 * NBUF:3 * NBUF]
    agg_sh = rest[3 * NBUF]
    xsem = rest[3 * NBUF + 1]
    isem = rest[3 * NBUF + 2:3 * NBUF + 2 + NBUF]
    ssem = rest[3 * NBUF + 2 + NBUF:3 * NBUF + 2 + 2 * NBUF]
    cid = lax.axis_index("c")
    sid = lax.axis_index("s")
    wid = sid * NC + cid

    xcopy = pltpu.async_copy(x_hbm, x_v, xsem)

    valsA = vals[0]

    def _z(i, _):
        valsA[pl.ds(i * L, L)] = jnp.zeros((L,), jnp.float32)
        return 0
    lax.fori_loop(0, CHUNK // L, _z, 0)
    z0 = sid * PER_TILE_NP
    for p in range(PER_TILE_NP // CHUNK):
        pltpu.sync_copy(valsA, agg_sh.at[pl.ds(z0 + p * CHUNK, CHUNK)])
    _rem = PER_TILE_NP - (PER_TILE_NP // CHUNK) * CHUNK
    pltpu.sync_copy(valsA.at[pl.ds(0, _rem)],
                    agg_sh.at[pl.ds(z0 + PER_TILE_NP - _rem, _rem)])

    start = wid * CNT + jnp.minimum(wid, EXTRA2)

    cnt_w = jnp.where(wid < EXTRA2, CNT + 1, CNT)

    def issue_idx(c, b):
        e0 = c * CHUNK
        pltpu.async_copy(edge_hbm.at[0, pl.ds(e0, CHUNK)], src[b], isem[b])
        pltpu.async_copy(edge_hbm.at[1, pl.ds(e0, CHUNK)], dst[b], isem[b])

    def wait_idx(c, b):
        e0 = c * CHUNK
        pltpu.make_async_copy(edge_hbm.at[0, pl.ds(e0, CHUNK)], src[b], isem[b]).wait()
        pltpu.make_async_copy(edge_hbm.at[1, pl.ds(e0, CHUNK)], dst[b], isem[b]).wait()

    def start_scatter(b):
        return pltpu.async_copy(vals[b], agg_sh.at[dst[b]], ssem[b], add=True)

    def drain_scatter(b):
        pltpu.make_async_copy(vals[b], agg_sh.at[dst[b]], ssem[b]).wait()

    def gather_loop(b):
        src_v, vals_v = src[b], vals[b]

        @plsc.parallel_loop(0, CHUNK // L, unroll=U)
        def _g(i):
            off = i * L
            idx = src_v[pl.ds(off, L)]
            vals_v[pl.ds(off, L)] = plsc.load_gather(x_v, [idx])

    for b in range(LA):
        issue_idx(start + b, b)
    xcopy.wait()
    plsc.subcore_barrier()

    def _group(g, _):
        j0 = NBUF * g
        for b in range(NBUF):
            j = j0 + b
            c = start + j
            wait_idx(c, b)
            gather_loop(b)
            start_scatter(b)
            pb = (b + LA) % NBUF
            if b == 0:
                @pl.when(g > 0)
                def _():
                    drain_scatter(pb)
            else:
                drain_scatter(pb)

            @pl.when(j + LA < cnt_w)
            def _():
                issue_idx(c + LA, pb)
        return 0
    lax.fori_loop(0, GROUPS, _group, 0)

    j96 = NBUF * GROUPS
    wait_idx(start + j96, 0)
    gather_loop(0)
    start_scatter(0)

    @pl.when(wid < EXTRA2)
    def _():
        wait_idx(start + j96 + 1, 1)
        gather_loop(1)
        start_scatter(1).wait()
    drain_scatter(NBUF - 1)
    drain_scatter(0)

    plsc.subcore_barrier()
    for p in range(PER_TILE_NP // CHUNK):
        pltpu.sync_copy(agg_sh.at[pl.ds(z0 + p * CHUNK, CHUNK)], valsA)
        pltpu.sync_copy(valsA, aggp_hbm.at[cid, pl.ds(z0 + p * CHUNK, CHUNK)])
    pltpu.sync_copy(agg_sh.at[pl.ds(z0 + PER_TILE_NP - _rem, _rem)],
                    valsA.at[pl.ds(0, _rem)])
    pltpu.sync_copy(valsA.at[pl.ds(0, _rem)],
                    aggp_hbm.at[cid, pl.ds(z0 + PER_TILE_NP - _rem, _rem)])


@functools.partial(
    pl.kernel,
    out_type=(jax.ShapeDtypeStruct((NC, 128), jnp.float32),
              jax.ShapeDtypeStruct((NC, 128), jnp.float32)),
    mesh=_mesh,
    scratch_types=[
        pltpu.VMEM((PER_W_NP,), jnp.float32),
        pltpu.VMEM((PER_W_NP,), jnp.float32),
        pltpu.VMEM((PER_W_NP,), jnp.float32),
        pltpu.VMEM((PER_W_NP,), jnp.float32),
        pltpu.VMEM((PER_W_NP,), jnp.int32),
        pltpu.VMEM((128,), jnp.float32),
        pltpu.VMEM_SHARED((128,), jnp.float32),
        pltpu.VMEM_SHARED((128,), jnp.float32),
        pltpu.SemaphoreType.DMA,
        pltpu.SemaphoreType.DMA,
    ],
)
def _pool_kernel(aggp_hbm, batch_hbm, sums_hbm, cnts_hbm,
                 a0_v, a1_v, h_v, ones_v, bidx_v, stage_v,
                 sums_sh, cnts_sh, hsem, csem):
    cid = lax.axis_index("c")
    sid = lax.axis_index("s")
    wid = cid * NS + sid

    for i in range(128 // L):
        stage_v[pl.ds(i * L, L)] = jnp.zeros((L,), jnp.float32)

    @pl.when(sid == 0)
    def _():
        pltpu.sync_copy(stage_v, sums_sh)
        pltpu.sync_copy(stage_v, cnts_sh)
    plsc.subcore_barrier()

    base = wid * PER_W_NP
    pltpu.sync_copy(aggp_hbm.at[0, pl.ds(base, PER_W_NP)], a0_v)
    pltpu.sync_copy(aggp_hbm.at[1, pl.ds(base, PER_W_NP)], a1_v)
    pltpu.sync_copy(batch_hbm.at[pl.ds(base, PER_W_NP)], bidx_v)

    def _relu(i, _):
        off = i * L
        a = a0_v[pl.ds(off, L)] + a1_v[pl.ds(off, L)]
        h_v[pl.ds(off, L)] = jnp.maximum(a, 0.0)
        ones_v[pl.ds(off, L)] = jnp.ones((L,), jnp.float32)
        return 0
    lax.fori_loop(0, PER_W_NP // L, _relu, 0)

    pltpu.async_copy(h_v, sums_sh.at[bidx_v], hsem, add=True).wait()
    pltpu.async_copy(ones_v, cnts_sh.at[bidx_v], csem, add=True).wait()

    plsc.subcore_barrier()

    @pl.when(sid == 0)
    def _():
        pltpu.sync_copy(sums_sh, stage_v)
        pltpu.sync_copy(stage_v, sums_hbm.at[cid])
        pltpu.sync_copy(cnts_sh, stage_v)
        pltpu.sync_copy(stage_v, cnts_hbm.at[cid])


def kernel(x, edge_index, batch, W, b):
    ei = edge_index.astype(jnp.int32)
    batch_p = jnp.concatenate(
        [batch.astype(jnp.int32),
         jnp.full((NP - N_NODES,), N_GRAPHS, jnp.int32)])

    aggp = _edge_kernel(ei, x.reshape(-1))
    sums, cnts = _pool_kernel(aggp, batch_p)

    tot_s = sums[0] + sums[1]
    tot_c = cnts[0] + cnts[1]
    pooled = tot_s[:N_GRAPHS] / jnp.maximum(tot_c[:N_GRAPHS], 1.0)
    return pooled[:, None] * W[0, 0] + b


# speedup vs baseline: 651.4738x; 1.0137x over previous
import functools

import jax
import jax.numpy as jnp
from jax import lax
from jax.experimental import pallas as pl
from jax.experimental.pallas import tpu as pltpu
from jax.experimental.pallas import tpu_sc as plsc

N_NODES = 100000
N_EDGES = 6400000
N_GRAPHS = 64

NC = 2
NS = 16
NW = NC * NS
L = 16

NP = 102400
PER_TILE_NP = NP // NS
PER_W_NP = NP // NW

CHUNK = 2048
NCH = N_EDGES // CHUNK
CNT = 97
EXTRA2 = NCH - NW * CNT
NBUF = 4
LA = 3
GROUPS = 24
U = 8

_mesh = plsc.VectorSubcoreMesh(core_axis_name="c", subcore_axis_name="s")


@functools.partial(
    pl.kernel,
    out_type=jax.ShapeDtypeStruct((NC, NP), jnp.float32),
    mesh=_mesh,
    scratch_types=[
        pltpu.VMEM((N_NODES,), jnp.float32),
    ] + [pltpu.VMEM((CHUNK,), jnp.int32) for _ in range(NBUF)]
      + [pltpu.VMEM((CHUNK,), jnp.int32) for _ in range(NBUF)]
      + [pltpu.VMEM((CHUNK,), jnp.float32) for _ in range(NBUF)]
      + [pltpu.VMEM_SHARED((NP,), jnp.float32)]
      + [pltpu.SemaphoreType.DMA]
      + [pltpu.SemaphoreType.DMA for _ in range(NBUF)]
      + [pltpu.SemaphoreType.DMA for _ in range(NBUF)],
    compiler_params=pltpu.CompilerParams(needs_layout_passes=False),
)
def _edge_kernel(edge_hbm, x_hbm, aggp_hbm, x_v, *rest):
    src = rest[0:NBUF]
    dst = rest[NBUF:2 * NBUF]
    vals = rest[2 * NBUF:3 * NBUF]
    agg_sh = rest[3 * NBUF]
    xsem = rest[3 * NBUF + 1]
    isem = rest[3 * NBUF + 2:3 * NBUF + 2 + NBUF]
    ssem = rest[3 * NBUF + 2 + NBUF:3 * NBUF + 2 + 2 * NBUF]
    cid = lax.axis_index("c")
    sid = lax.axis_index("s")
    wid = sid * NC + cid

    xcopy = pltpu.async_copy(x_hbm, x_v, xsem)

    valsA = vals[0]

    def _z(i, _):
        valsA[pl.ds(i * L, L)] = jnp.zeros((L,), jnp.float32)
        return 0
    lax.fori_loop(0, CHUNK // L, _z, 0)
    z0 = sid * PER_TILE_NP
    _rem = PER_TILE_NP - (PER_TILE_NP // CHUNK) * CHUNK
    zs = [pltpu.async_copy(valsA, agg_sh.at[pl.ds(z0 + p * CHUNK, CHUNK)],
                           ssem[p]) for p in range(PER_TILE_NP // CHUNK)]
    zs.append(pltpu.async_copy(valsA.at[pl.ds(0, _rem)],
                               agg_sh.at[pl.ds(z0 + PER_TILE_NP - _rem, _rem)],
                               ssem[3]))
    for z in zs:
        z.wait()

    start = wid * CNT + jnp.minimum(wid, EXTRA2)

    cnt_w = jnp.where(wid < EXTRA2, CNT + 1, CNT)

    def issue_idx(c, b):
        e0 = c * CHUNK
        pltpu.async_copy(edge_hbm.at[0, pl.ds(e0, CHUNK)], src[b], isem[b])
        pltpu.async_copy(edge_hbm.at[1, pl.ds(e0, CHUNK)], dst[b], isem[b])

    def wait_idx(c, b):
        e0 = c * CHUNK
        pltpu.make_async_copy(edge_hbm.at[0, pl.ds(e0, CHUNK)], src[b], isem[b]).wait()
        pltpu.make_async_copy(edge_hbm.at[1, pl.ds(e0, CHUNK)], dst[b], isem[b]).wait()

    def start_scatter(b):
        return pltpu.async_copy(vals[b], agg_sh.at[dst[b]], ssem[b], add=True)

    def drain_scatter(b):
        pltpu.make_async_copy(vals[b], agg_sh.at[dst[b]], ssem[b]).wait()

    def gather_loop(b):
        src_v, vals_v = src[b], vals[b]

        @plsc.parallel_loop(0, CHUNK // L, unroll=U)
        def _g(i):
            off = i * L
            idx = src_v[pl.ds(off, L)]
            vals_v[pl.ds(off, L)] = plsc.load_gather(x_v, [idx])

    for b in range(LA):
        issue_idx(start + b, b)
    xcopy.wait()
    plsc.subcore_barrier()

    def _group(g, _):
        j0 = NBUF * g
        for b in range(NBUF):
            j = j0 + b
            c = start + j
            wait_idx(c, b)
            gather_loop(b)
            start_scatter(b)
            pb = (b + LA) % NBUF
            if b == 0:
                @pl.when(g > 0)
                def _():
                    drain_scatter(pb)
            else:
                drain_scatter(pb)

            @pl.when(j + LA < cnt_w)
            def _():
                issue_idx(c + LA, pb)
        return 0
    lax.fori_loop(0, GROUPS, _group, 0)

    j96 = NBUF * GROUPS
    wait_idx(start + j96, 0)
    gather_loop(0)
    start_scatter(0)

    @pl.when(wid < EXTRA2)
    def _():
        wait_idx(start + j96 + 1, 1)
        gather_loop(1)
        start_scatter(1).wait()
    drain_scatter(NBUF - 1)
    drain_scatter(0)

    plsc.subcore_barrier()
    obuf = [vals[0], vals[1], vals[2], vals[3]]
    o1 = [pltpu.async_copy(agg_sh.at[pl.ds(z0 + p * CHUNK, CHUNK)], obuf[p],
                           isem[p]) for p in range(PER_TILE_NP // CHUNK)]
    o1.append(pltpu.async_copy(agg_sh.at[pl.ds(z0 + PER_TILE_NP - _rem, _rem)],
                               obuf[3].at[pl.ds(0, _rem)], isem[3]))
    for o in o1:
        o.wait()
    o2 = [pltpu.async_copy(obuf[p], aggp_hbm.at[cid, pl.ds(z0 + p * CHUNK, CHUNK)],
                           ssem[p]) for p in range(PER_TILE_NP // CHUNK)]
    o2.append(pltpu.async_copy(obuf[3].at[pl.ds(0, _rem)],
                               aggp_hbm.at[cid, pl.ds(z0 + PER_TILE_NP - _rem, _rem)],
                               ssem[3]))
    for o in o2:
        o.wait()


@functools.partial(
    pl.kernel,
    out_type=(jax.ShapeDtypeStruct((NC, 128), jnp.float32),
              jax.ShapeDtypeStruct((NC, 128), jnp.float32)),
    mesh=_mesh,
    scratch_types=[
        pltpu.VMEM((PER_W_NP,), jnp.float32),
        pltpu.VMEM((PER_W_NP,), jnp.float32),
        pltpu.VMEM((PER_W_NP,), jnp.float32),
        pltpu.VMEM((PER_W_NP,), jnp.float32),
        pltpu.VMEM((PER_W_NP,), jnp.int32),
        pltpu.VMEM((128,), jnp.float32),
        pltpu.VMEM_SHARED((128,), jnp.float32),
        pltpu.VMEM_SHARED((128,), jnp.float32),
        pltpu.SemaphoreType.DMA,
        pltpu.SemaphoreType.DMA,
        pltpu.SemaphoreType.DMA,
    ],
)
def _pool_kernel(aggp_hbm, batch_hbm, sums_hbm, cnts_hbm,
                 a0_v, a1_v, h_v, ones_v, bidx_v, stage_v,
                 sums_sh, cnts_sh, hsem, csem, insem):
    cid = lax.axis_index("c")
    sid = lax.axis_index("s")
    wid = cid * NS + sid

    for i in range(128 // L):
        stage_v[pl.ds(i * L, L)] = jnp.zeros((L,), jnp.float32)

    @pl.when(sid == 0)
    def _():
        pltpu.sync_copy(stage_v, sums_sh)
        pltpu.sync_copy(stage_v, cnts_sh)
    plsc.subcore_barrier()

    base = wid * PER_W_NP
    i0 = pltpu.async_copy(aggp_hbm.at[0, pl.ds(base, PER_W_NP)], a0_v, insem)
    i1 = pltpu.async_copy(aggp_hbm.at[1, pl.ds(base, PER_W_NP)], a1_v, insem)
    i2 = pltpu.async_copy(batch_hbm.at[pl.ds(base, PER_W_NP)], bidx_v, insem)
    i0.wait()
    i1.wait()
    i2.wait()

    @plsc.parallel_loop(0, PER_W_NP // L, unroll=8)
    def _relu(i):
        off = i * L
        a = a0_v[pl.ds(off, L)] + a1_v[pl.ds(off, L)]
        h_v[pl.ds(off, L)] = jnp.maximum(a, 0.0)
        ones_v[pl.ds(off, L)] = jnp.ones((L,), jnp.float32)

    sh = pltpu.async_copy(h_v, sums_sh.at[bidx_v], hsem, add=True)
    sc = pltpu.async_copy(ones_v, cnts_sh.at[bidx_v], csem, add=True)
    sh.wait()
    sc.wait()

    plsc.subcore_barrier()

    @pl.when(sid == 0)
    def _():
        pltpu.sync_copy(sums_sh, stage_v)
        pltpu.sync_copy(stage_v, sums_hbm.at[cid])
        pltpu.sync_copy(cnts_sh, stage_v)
        pltpu.sync_copy(stage_v, cnts_hbm.at[cid])


def kernel(x, edge_index, batch, W, b):
    ei = edge_index.astype(jnp.int32)
    batch_p = jnp.concatenate(
        [batch.astype(jnp.int32),
         jnp.full((NP - N_NODES,), N_GRAPHS, jnp.int32)])

    aggp = _edge_kernel(ei, x.reshape(-1))
    sums, cnts = _pool_kernel(aggp, batch_p)

    tot_s = sums[0] + sums[1]
    tot_c = cnts[0] + cnts[1]
    pooled = tot_s[:N_GRAPHS] / jnp.maximum(tot_c[:N_GRAPHS], 1.0)
    return pooled[:, None] * W[0, 0] + b
